# baseline (speedup 1.0000x reference)
import functools

import jax
import jax.numpy as jnp
from jax import lax
from jax.experimental import pallas as pl
from jax.experimental.pallas import tpu as pltpu
from jax.experimental.pallas import tpu_sc as plsc

N_NODES = 10000
N_EDGES = 320000
D_NODE = 128
D_EDGE = 16
LANES = 16

NC = 2
NS = 16
NW = NC * NS
CHUNK = 128
E_PAD = 327680
K_PER_W = E_PAD // (NW * CHUNK)
GRP = 8
N_GRP = K_PER_W // GRP
PK = CHUNK // 8
PK_ROWS = E_PAD // 8
MLP_ROWS = N_EDGES // 8
R_BLK = 1000


def _proj_body(x_ref, w_ref, ts_ref, td_ref):
    x = x_ref[...]
    w = w_ref[...]
    ts_ref[...] = jnp.dot(x, w[:, :LANES], preferred_element_type=jnp.float32)
    td_ref[...] = jnp.dot(x, w[:, LANES:], preferred_element_type=jnp.float32)


def _project(x, w_sd):
    return pl.pallas_call(
        _proj_body,
        out_shape=(
            jax.ShapeDtypeStruct((N_NODES, LANES), jnp.float32),
            jax.ShapeDtypeStruct((N_NODES, LANES), jnp.float32),
        ),
    )(x, w_sd)


def _gather_body(ts_hbm, td_hbm, src_hbm, dst_hbm, gs_hbm,
                 idxs_v, idxd_v, rbufs, pbufs, sem_g, sem_a, sem_w):
    wid = lax.axis_index("s") * NC + lax.axis_index("c")
    c0 = wid * K_PER_W
    pltpu.sync_copy(src_hbm.at[pl.ds(c0, K_PER_W)], idxs_v)
    pltpu.sync_copy(dst_hbm.at[pl.ds(c0, K_PER_W)], idxd_v)

    rs = [rbufs.at[k] for k in range(GRP)]
    ps = [pbufs.at[k] for k in range(GRP)]

    def fire_s(lc, k):
        pltpu.async_copy(ts_hbm.at[idxs_v.at[lc]], rs[k], sem_g)

    def wait_s(lc, k):
        pltpu.make_async_copy(ts_hbm.at[idxs_v.at[lc]], rs[k], sem_g).wait()

    def fire_d(lc, k):
        pltpu.async_copy(td_hbm.at[idxd_v.at[lc]], rs[k], sem_a, add=True)

    def wait_d(lc, k):
        pltpu.make_async_copy(td_hbm.at[idxd_v.at[lc]], rs[k], sem_a).wait()

    def repack(src_buf, dst_buf):
        for j in range(PK):
            for m in range(8):
                dst_buf[j, pl.ds(16 * m, 16)] = src_buf[8 * j + m, :]

    def write_pk(lc, k):
        r0 = (c0 + lc) * PK
        pltpu.async_copy(ps[k], gs_hbm.at[pl.ds(r0, PK)], sem_w)

    def drain_pk(lc, k):
        r0 = (c0 + lc) * PK
        pltpu.make_async_copy(ps[k], gs_hbm.at[pl.ds(r0, PK)], sem_w).wait()

    def body(i, carry):
        base = i * GRP
        for k in range(GRP):
            fire_s(base + k, k)
        for k in range(GRP):
            wait_s(base + k, k)
            fire_d(base + k, k)
        for k in range(GRP):
            wait_d(base + k, k)

            @pl.when(i > 0)
            def _():
                drain_pk(base - GRP + k, k)

            repack(rs[k], ps[k])
            write_pk(base + k, k)
        return carry

    lax.fori_loop(0, N_GRP, body, 0)
    for k in range(GRP):
        drain_pk((N_GRP - 1) * GRP + k, k)


def _gather(table_s, table_d, src2, dst2):
    mesh = plsc.VectorSubcoreMesh(core_axis_name="c", subcore_axis_name="s")
    f = functools.partial(
        pl.kernel,
        mesh=mesh,
        compiler_params=pltpu.CompilerParams(use_tc_tiling_on_sc=False),
        out_type=jax.ShapeDtypeStruct((PK_ROWS, 128), jnp.float32),
        scratch_types=[
            pltpu.VMEM((K_PER_W, CHUNK), jnp.int32),
            pltpu.VMEM((K_PER_W, CHUNK), jnp.int32),
            pltpu.VMEM((GRP, CHUNK, LANES), jnp.float32),
            pltpu.VMEM((GRP, PK, 128), jnp.float32),
            pltpu.SemaphoreType.DMA,
            pltpu.SemaphoreType.DMA,
            pltpu.SemaphoreType.DMA,
        ],
    )(_gather_body)
    return f(table_s, table_d, src2, dst2)


def _mlp_body(gs_ref, e_ref, w7c_ref, b7_ref, w8_ref, b8_ref,
              w81_ref, b81_ref, w9_ref, b9_ref, out_ref):
    h = gs_ref[...] + jnp.dot(e_ref[...], w7c_ref[...],
                              preferred_element_type=jnp.float32) + b7_ref[...]
    h = jnp.where(h >= 0, h, 0.1 * h)
    h = jnp.dot(h, w8_ref[...], preferred_element_type=jnp.float32) + b8_ref[...]
    h = jnp.where(h >= 0, h, 0.1 * h)
    h = jnp.dot(h, w81_ref[...], preferred_element_type=jnp.float32) + b81_ref[...]
    h = jnp.where(h >= 0, h, 0.1 * h)
    h = jnp.dot(h, w9_ref[...], preferred_element_type=jnp.float32) + b9_ref[...]
    hm1 = jnp.roll(h, -1, axis=1)
    hp1 = jnp.roll(h, 1, axis=1)
    lane = jax.lax.broadcasted_iota(jnp.int32, h.shape, 1)
    diff = jnp.where(lane % LANES == 0, hm1 - h, hp1 - h)
    out_ref[...] = 1.0 / (1.0 + jnp.exp(diff))


def _mlp(gs, e_pk, w7c, b7p, w8p, b8p, w81p, b81p, w9p, b9p):
    n_blocks = MLP_ROWS // R_BLK
    eb = lambda i: (i, 0)
    wb = lambda i: (0, 0)
    return pl.pallas_call(
        _mlp_body,
        grid=(n_blocks,),
        in_specs=[
            pl.BlockSpec((R_BLK, 128), eb),
            pl.BlockSpec((R_BLK, 128), eb),
            pl.BlockSpec((128, 128), wb),
            pl.BlockSpec((1, 128), wb),
            pl.BlockSpec((128, 128), wb),
            pl.BlockSpec((1, 128), wb),
            pl.BlockSpec((128, 128), wb),
            pl.BlockSpec((1, 128), wb),
            pl.BlockSpec((128, 128), wb),
            pl.BlockSpec((1, 128), wb),
        ],
        out_specs=pl.BlockSpec((R_BLK, 128), eb),
        out_shape=jax.ShapeDtypeStruct((MLP_ROWS, 128), jnp.float32),
    )(gs, e_pk, w7c, b7p, w8p, b8p, w81p, b81p, w9p, b9p)


def _pad16(w):
    out = jnp.zeros((LANES, LANES), jnp.float32)
    return out.at[: w.shape[0], : w.shape[1]].set(w)


def _bdiag(w16):
    return jnp.kron(jnp.eye(8, dtype=jnp.float32), w16)


def _btile(b):
    v = jnp.zeros((LANES,), jnp.float32).at[: b.shape[0]].set(b)
    return jnp.tile(v, 8)[None, :]


def kernel(x, e, edge_index, W7, b7, W8, b8, W81, b81, W9, b9):
    w_sd = jnp.zeros((D_NODE, 2 * LANES), jnp.float32)
    w_sd = w_sd.at[:, 0:6].set(W7[0:D_NODE])
    w_sd = w_sd.at[:, LANES:LANES + 6].set(W7[D_NODE:2 * D_NODE])
    w7c = _bdiag(_pad16(W7[2 * D_NODE:]))
    w8p = _bdiag(_pad16(W8))
    w81p = _bdiag(_pad16(W81))
    w9p = _bdiag(_pad16(W9))
    b7p = _btile(b7)
    b8p = _btile(b8)
    b81p = _btile(b81)
    b9p = _btile(b9)

    idx = edge_index.astype(jnp.int32)
    pad = jnp.zeros((E_PAD - N_EDGES,), jnp.int32)
    src2 = jnp.concatenate([idx[0], pad]).reshape(E_PAD // CHUNK, CHUNK)
    dst2 = jnp.concatenate([idx[1], pad]).reshape(E_PAD // CHUNK, CHUNK)

    table_s, table_d = _project(x, w_sd)
    gs = _gather(table_s, table_d, src2, dst2)

    e_pk = e.reshape(MLP_ROWS, 128)
    out = _mlp(gs, e_pk, w7c, b7p, w8p, b8p, w81p, b81p, w9p, b9p)
    p0 = out[:, 0::LANES]
    p1 = out[:, 1::LANES]
    return jnp.stack([p0, p1], axis=-1).reshape(N_EDGES, 2)


# speedup vs baseline: 1.2157x; 1.2157x over previous
import functools

import jax
import jax.numpy as jnp
from jax import lax
from jax.experimental import pallas as pl
from jax.experimental.pallas import tpu as pltpu
from jax.experimental.pallas import tpu_sc as plsc

N_NODES = 10000
N_EDGES = 320000
D_NODE = 128
D_EDGE = 16
LANES = 16

NC = 2
NS = 16
NW = NC * NS
CHUNK = 128
E_PAD = 327680
K_PER_W = E_PAD // (NW * CHUNK)
GRP = 4
N_GRP = K_PER_W // GRP
PK = CHUNK // 8
PK_ROWS = E_PAD // 8
MLP_ROWS = N_EDGES // 8
R_BLK = 1000


def _proj_body(x_ref, w_ref, ts_ref, td_ref):
    x = x_ref[...]
    w = w_ref[...]
    ts_ref[...] = jnp.dot(x, w[:, :LANES], preferred_element_type=jnp.float32)
    td_ref[...] = jnp.dot(x, w[:, LANES:], preferred_element_type=jnp.float32)


def _project(x, w_sd):
    return pl.pallas_call(
        _proj_body,
        out_shape=(
            jax.ShapeDtypeStruct((N_NODES, LANES), jnp.float32),
            jax.ShapeDtypeStruct((N_NODES, LANES), jnp.float32),
        ),
    )(x, w_sd)


def _gather_body(ts_hbm, td_hbm, src_hbm, dst_hbm, gs_hbm, gd_hbm,
                 idxs_v, idxd_v, rbufs, pbufs, sem_g, sem_w):
    wid = lax.axis_index("s") * NC + lax.axis_index("c")
    c0 = wid * K_PER_W
    pltpu.sync_copy(src_hbm.at[pl.ds(c0, K_PER_W)], idxs_v)
    pltpu.sync_copy(dst_hbm.at[pl.ds(c0, K_PER_W)], idxd_v)

    rs = [rbufs.at[k] for k in range(GRP)]
    rd = [rbufs.at[GRP + k] for k in range(GRP)]
    ps = [pbufs.at[k] for k in range(GRP)]
    pd = [pbufs.at[GRP + k] for k in range(GRP)]

    def fire(lc, k):
        pltpu.async_copy(ts_hbm.at[idxs_v.at[lc]], rs[k], sem_g)
        pltpu.async_copy(td_hbm.at[idxd_v.at[lc]], rd[k], sem_g)

    def wait_gather(lc, k):
        pltpu.make_async_copy(ts_hbm.at[idxs_v.at[lc]], rs[k], sem_g).wait()
        pltpu.make_async_copy(td_hbm.at[idxd_v.at[lc]], rd[k], sem_g).wait()

    def repack(src_buf, dst_buf):
        for j in range(PK):
            for m in range(8):
                dst_buf[j, pl.ds(16 * m, 16)] = src_buf[8 * j + m, :]

    def write_pk(lc, k):
        r0 = (c0 + lc) * PK
        pltpu.async_copy(ps[k], gs_hbm.at[pl.ds(r0, PK)], sem_w)
        pltpu.async_copy(pd[k], gd_hbm.at[pl.ds(r0, PK)], sem_w)

    def drain_pk(lc, k):
        r0 = (c0 + lc) * PK
        pltpu.make_async_copy(ps[k], gs_hbm.at[pl.ds(r0, PK)], sem_w).wait()
        pltpu.make_async_copy(pd[k], gd_hbm.at[pl.ds(r0, PK)], sem_w).wait()

    def body(i, carry):
        base = i * GRP
        for k in range(GRP):
            fire(base + k, k)
        for k in range(GRP):
            wait_gather(base + k, k)

            @pl.when(i > 0)
            def _():
                drain_pk(base - GRP + k, k)

            repack(rs[k], ps[k])
            repack(rd[k], pd[k])
            write_pk(base + k, k)
        return carry

    lax.fori_loop(0, N_GRP, body, 0)
    for k in range(GRP):
        drain_pk((N_GRP - 1) * GRP + k, k)


def _gather(table_s, table_d, src2, dst2):
    mesh = plsc.VectorSubcoreMesh(core_axis_name="c", subcore_axis_name="s")
    f = functools.partial(
        pl.kernel,
        mesh=mesh,
        compiler_params=pltpu.CompilerParams(use_tc_tiling_on_sc=False),
        out_type=[
            jax.ShapeDtypeStruct((PK_ROWS, 128), jnp.float32),
            jax.ShapeDtypeStruct((PK_ROWS, 128), jnp.float32),
        ],
        scratch_types=[
            pltpu.VMEM((K_PER_W, CHUNK), jnp.int32),
            pltpu.VMEM((K_PER_W, CHUNK), jnp.int32),
            pltpu.VMEM((2 * GRP, CHUNK, LANES), jnp.float32),
            pltpu.VMEM((2 * GRP, PK, 128), jnp.float32),
            pltpu.SemaphoreType.DMA,
            pltpu.SemaphoreType.DMA,
        ],
    )(_gather_body)
    return f(table_s, table_d, src2, dst2)


def _mlp_body(gs_ref, gd_ref, e_ref, w7c_ref, b7_ref, w8_ref, b8_ref,
              w81_ref, b81_ref, w9_ref, b9_ref, out_ref):
    h = gs_ref[...] + gd_ref[...]
    h = h + jnp.dot(e_ref[...], w7c_ref[...],
                    preferred_element_type=jnp.float32) + b7_ref[...]
    h = jnp.where(h >= 0, h, 0.1 * h)
    h = jnp.dot(h, w8_ref[...], preferred_element_type=jnp.float32) + b8_ref[...]
    h = jnp.where(h >= 0, h, 0.1 * h)
    h = jnp.dot(h, w81_ref[...], preferred_element_type=jnp.float32) + b81_ref[...]
    h = jnp.where(h >= 0, h, 0.1 * h)
    h = jnp.dot(h, w9_ref[...], preferred_element_type=jnp.float32) + b9_ref[...]
    hm1 = jnp.roll(h, -1, axis=1)
    hp1 = jnp.roll(h, 1, axis=1)
    lane = jax.lax.broadcasted_iota(jnp.int32, h.shape, 1)
    diff = jnp.where(lane % LANES == 0, hm1 - h, hp1 - h)
    out_ref[...] = 1.0 / (1.0 + jnp.exp(diff))


def _mlp(gs, gd, e_pk, w7c, b7p, w8p, b8p, w81p, b81p, w9p, b9p):
    n_blocks = MLP_ROWS // R_BLK
    eb = lambda i: (i, 0)
    wb = lambda i: (0, 0)
    return pl.pallas_call(
        _mlp_body,
        grid=(n_blocks,),
        in_specs=[
            pl.BlockSpec((R_BLK, 128), eb),
            pl.BlockSpec((R_BLK, 128), eb),
            pl.BlockSpec((R_BLK, 128), eb),
            pl.BlockSpec((128, 128), wb),
            pl.BlockSpec((1, 128), wb),
            pl.BlockSpec((128, 128), wb),
            pl.BlockSpec((1, 128), wb),
            pl.BlockSpec((128, 128), wb),
            pl.BlockSpec((1, 128), wb),
            pl.BlockSpec((128, 128), wb),
            pl.BlockSpec((1, 128), wb),
        ],
        out_specs=pl.BlockSpec((R_BLK, 128), eb),
        out_shape=jax.ShapeDtypeStruct((MLP_ROWS, 128), jnp.float32),
    )(gs, gd, e_pk, w7c, b7p, w8p, b8p, w81p, b81p, w9p, b9p)


def _pad16(w):
    out = jnp.zeros((LANES, LANES), jnp.float32)
    return out.at[: w.shape[0], : w.shape[1]].set(w)


def _bdiag(w16):
    return jnp.kron(jnp.eye(8, dtype=jnp.float32), w16)


def _btile(b):
    v = jnp.zeros((LANES,), jnp.float32).at[: b.shape[0]].set(b)
    return jnp.tile(v, 8)[None, :]


def kernel(x, e, edge_index, W7, b7, W8, b8, W81, b81, W9, b9):
    w_sd = jnp.zeros((D_NODE, 2 * LANES), jnp.float32)
    w_sd = w_sd.at[:, 0:6].set(W7[0:D_NODE])
    w_sd = w_sd.at[:, LANES:LANES + 6].set(W7[D_NODE:2 * D_NODE])
    w7c = _bdiag(_pad16(W7[2 * D_NODE:]))
    w8p = _bdiag(_pad16(W8))
    w81p = _bdiag(_pad16(W81))
    w9p = _bdiag(_pad16(W9))
    b7p = _btile(b7)
    b8p = _btile(b8)
    b81p = _btile(b81)
    b9p = _btile(b9)

    idx = edge_index.astype(jnp.int32)
    pad = jnp.zeros((E_PAD - N_EDGES,), jnp.int32)
    src2 = jnp.concatenate([idx[0], pad]).reshape(E_PAD // CHUNK, CHUNK)
    dst2 = jnp.concatenate([idx[1], pad]).reshape(E_PAD // CHUNK, CHUNK)

    table_s, table_d = _project(x, w_sd)
    gs, gd = _gather(table_s, table_d, src2, dst2)

    e_pk = e.reshape(MLP_ROWS, 128)
    out = _mlp(gs, gd, e_pk, w7c, b7p, w8p, b8p, w81p, b81p, w9p, b9p)
    return out.reshape(N_EDGES, LANES)[:, :2]
